# baseline (speedup 1.0000x reference)
import functools

import jax
import jax.numpy as jnp
from jax import lax
from jax.experimental import pallas as pl
from jax.experimental.pallas import tpu as pltpu
from jax.experimental.pallas import tpu_sc as plsc

_NC = 2
_NS = 16
_NW = _NC * _NS
_L = 16
_CRI = 16
_CRO = 8
_NIN = 2
_NOUT = 2
_H = _CRI // _CRO


@functools.lru_cache(maxsize=None)
def _build(R, D):
    assert R % (_NW * _CRI * _NIN) == 0 and D % _L == 0
    rows_per_w = R // _NW
    n_chunks = rows_per_w // _CRI
    n_rounds = n_chunks // _NIN
    n_gran = D // _L

    mesh = plsc.VectorSubcoreMesh(core_axis_name="c", subcore_axis_name="s")

    @functools.partial(
        pl.kernel,
        out_type=jax.ShapeDtypeStruct((R, D), jnp.float32),
        mesh=mesh,
        scratch_types=[
            pltpu.VMEM((D,), jnp.int32),
            *([pltpu.VMEM((_CRI, D), jnp.float32)] * _NIN),
            *([pltpu.VMEM((_CRO, D), jnp.float32)] * _NOUT),
            *([pltpu.SemaphoreType.DMA] * (_NIN + _NOUT)),
        ],
        compiler_params=pltpu.CompilerParams(needs_layout_passes=False),
    )
    def permute(in_hbm, idx_hbm, out_hbm, idx_v, *bufs):
        ins = bufs[:_NIN]
        outs = bufs[_NIN:_NIN + _NOUT]
        isems = bufs[_NIN + _NOUT:2 * _NIN + _NOUT]
        osems = bufs[2 * _NIN + _NOUT:]

        wid = lax.axis_index("s") * _NC + lax.axis_index("c")
        base = wid * rows_per_w
        last_row0 = base + (n_chunks - 1) * _CRI
        pltpu.sync_copy(idx_hbm, idx_v)

        rvecs = [jnp.full((_L,), r, jnp.int32) for r in range(_CRI)]

        def in_copy(row0, b):
            return pltpu.make_async_copy(
                in_hbm.at[pl.ds(row0, _CRI)], ins[b], isems[b]
            )

        def out_copy(row0, b):
            return pltpu.make_async_copy(
                outs[b], out_hbm.at[pl.ds(row0, _CRO)], osems[b]
            )

        for b in range(_NIN):
            in_copy(base + b * _CRI, b).start()

        @pl.loop(0, n_rounds)
        def round_(t):
            for k in range(_NIN):
                row0 = base + (t * _NIN + k) * _CRI
                in_copy(row0, k).wait()
                for h in range(_H):
                    ob = (k * _H + h) % _NOUT
                    orow0 = row0 + h * _CRO

                    if k * _H + h >= _NOUT:
                        out_copy(orow0, ob).wait()
                    else:
                        @pl.when(t > 0)
                        def _():
                            out_copy(orow0, ob).wait()

                    @plsc.parallel_loop(0, n_gran, unroll=4)
                    def gran(j):
                        off = pl.multiple_of(j * _L, _L)
                        vidx = idx_v[pl.ds(off, _L)]
                        for r in range(_CRO):
                            vals = plsc.load_gather(
                                ins[k], [rvecs[h * _CRO + r], vidx]
                            )
                            outs[ob][r, pl.ds(off, _L)] = vals

                    out_copy(orow0, ob).start()

                nxt = jnp.minimum(row0 + _NIN * _CRI, last_row0)
                in_copy(nxt, k).start()

        for b in range(_NIN):
            in_copy(last_row0, b).wait()
        for b in range(_NOUT):
            out_copy(last_row0, b).wait()

    return permute


def kernel(inputs, idxs):
    shape = inputs.shape
    D = shape[-1]
    x = inputs.reshape(-1, D)
    out = _build(x.shape[0], D)(x, idxs)
    return out.reshape(shape)


# speedup vs baseline: 1.0080x; 1.0080x over previous
import functools

import jax
import jax.numpy as jnp
from jax import lax
from jax.experimental import pallas as pl
from jax.experimental.pallas import tpu as pltpu
from jax.experimental.pallas import tpu_sc as plsc

_NC = 2
_NS = 16
_NW = _NC * _NS
_L = 16
_CR = 8
_NBUF = 2


@functools.lru_cache(maxsize=None)
def _build(R, D):
    assert R % (_NW * _CR * _NBUF) == 0 and D % _L == 0
    rows_per_w = R // _NW
    n_chunks = rows_per_w // _CR
    n_rounds = n_chunks // _NBUF
    n_gran = D // _L

    mesh = plsc.VectorSubcoreMesh(core_axis_name="c", subcore_axis_name="s")

    @functools.partial(
        pl.kernel,
        out_type=jax.ShapeDtypeStruct((R, D), jnp.float32),
        mesh=mesh,
        scratch_types=[
            pltpu.VMEM((D,), jnp.int32),
            *([pltpu.VMEM((_CR, D), jnp.float32)] * _NBUF),
            *([pltpu.VMEM((_CR, D), jnp.float32)] * _NBUF),
            *([pltpu.SemaphoreType.DMA] * (2 * _NBUF)),
        ],
        compiler_params=pltpu.CompilerParams(needs_layout_passes=False),
    )
    def permute(in_hbm, idx_hbm, out_hbm, idx_v, *bufs):
        ins = bufs[:_NBUF]
        outs = bufs[_NBUF:2 * _NBUF]
        isems = bufs[2 * _NBUF:3 * _NBUF]
        osems = bufs[3 * _NBUF:]

        wid = lax.axis_index("s") * _NC + lax.axis_index("c")
        base = wid * rows_per_w
        last_row0 = base + (n_chunks - 1) * _CR
        pltpu.sync_copy(idx_hbm, idx_v)

        rvecs = [jnp.full((_L,), r, jnp.int32) for r in range(_CR)]

        def in_copy(row0, b):
            return pltpu.make_async_copy(
                in_hbm.at[pl.ds(row0, _CR)], ins[b], isems[b]
            )

        def out_copy(row0, b):
            return pltpu.make_async_copy(
                outs[b], out_hbm.at[pl.ds(row0, _CR)], osems[b]
            )

        for b in range(_NBUF):
            in_copy(base + b * _CR, b).start()

        @pl.loop(0, n_rounds)
        def round_(t):
            for b in range(_NBUF):
                row0 = base + (t * _NBUF + b) * _CR
                in_copy(row0, b).wait()

                @pl.when(t > 0)
                def _():
                    out_copy(row0, b).wait()

                @plsc.parallel_loop(0, n_gran, unroll=4)
                def gran(j):
                    off = pl.multiple_of(j * _L, _L)
                    vidx = idx_v[pl.ds(off, _L)]
                    for r in range(_CR):
                        vals = plsc.load_gather(ins[b], [rvecs[r], vidx])
                        outs[b][r, pl.ds(off, _L)] = vals

                out_copy(row0, b).start()
                nxt = jnp.minimum(row0 + _NBUF * _CR, last_row0)
                in_copy(nxt, b).start()

        for b in range(_NBUF):
            in_copy(last_row0, b).wait()
            out_copy(last_row0, b).wait()

    return permute


def kernel(inputs, idxs):
    shape = inputs.shape
    D = shape[-1]
    x = inputs.reshape(-1, D)
    out = _build(x.shape[0], D)(x, idxs)
    return out.reshape(shape)


# speedup vs baseline: 1.0214x; 1.0133x over previous
import functools

import jax
import jax.numpy as jnp
from jax import lax
from jax.experimental import pallas as pl
from jax.experimental.pallas import tpu as pltpu
from jax.experimental.pallas import tpu_sc as plsc

_NC = 2
_NS = 16
_NW = _NC * _NS
_L = 16
_CR = 8
_NBUF = 2


@functools.lru_cache(maxsize=None)
def _build(R, D):
    assert R % (_NW * _CR * _NBUF) == 0 and D % _L == 0
    rows_per_w = R // _NW
    n_chunks = rows_per_w // _CR
    n_rounds = n_chunks // _NBUF
    n_gran = D // _L

    mesh = plsc.VectorSubcoreMesh(core_axis_name="c", subcore_axis_name="s")

    @functools.partial(
        pl.kernel,
        out_type=jax.ShapeDtypeStruct((R, D), jnp.float32),
        mesh=mesh,
        scratch_types=[
            pltpu.VMEM((D,), jnp.int32),
            *([pltpu.VMEM((_CR, D), jnp.float32)] * _NBUF),
            *([pltpu.VMEM((_CR, D), jnp.float32)] * _NBUF),
            *([pltpu.SemaphoreType.DMA] * (2 * _NBUF)),
        ],
        compiler_params=pltpu.CompilerParams(needs_layout_passes=False),
    )
    def permute(in_hbm, idx_hbm, out_hbm, idx_v, *bufs):
        ins = bufs[:_NBUF]
        outs = bufs[_NBUF:2 * _NBUF]
        isems = bufs[2 * _NBUF:3 * _NBUF]
        osems = bufs[3 * _NBUF:]

        wid = lax.axis_index("s") * _NC + lax.axis_index("c")
        base = wid * rows_per_w
        last_row0 = base + (n_chunks - 1) * _CR

        rvecs = [jnp.full((_L,), r, jnp.int32) for r in range(_CR)]

        def in_copy(row0, b):
            return pltpu.make_async_copy(
                in_hbm.at[pl.ds(row0, _CR)], ins[b], isems[b]
            )

        def out_copy(row0, b):
            return pltpu.make_async_copy(
                outs[b], out_hbm.at[pl.ds(row0, _CR)], osems[b]
            )

        for b in range(_NBUF):
            in_copy(base + b * _CR, b).start()
        pltpu.sync_copy(idx_hbm, idx_v)

        @pl.loop(0, n_rounds)
        def round_(t):
            for b in range(_NBUF):
                row0 = base + (t * _NBUF + b) * _CR
                in_copy(row0, b).wait()

                @pl.when(t > 0)
                def _():
                    out_copy(row0, b).wait()

                @plsc.parallel_loop(0, n_gran, unroll=4)
                def gran(j):
                    off = pl.multiple_of(j * _L, _L)
                    vidx = idx_v[pl.ds(off, _L)]
                    for r in range(_CR):
                        vals = plsc.load_gather(ins[b], [rvecs[r], vidx])
                        outs[b][r, pl.ds(off, _L)] = vals

                out_copy(row0, b).start()

                @pl.when(t < n_rounds - 1)
                def _():
                    in_copy(row0 + _NBUF * _CR, b).start()

        for b in range(_NBUF):
            out_copy(last_row0, b).wait()

    return permute


def kernel(inputs, idxs):
    shape = inputs.shape
    D = shape[-1]
    x = inputs.reshape(-1, D)
    out = _build(x.shape[0], D)(x, idxs)
    return out.reshape(shape)
